# baseline (speedup 1.0000x reference)
import functools

import jax
import jax.numpy as jnp
from jax import lax
from jax.experimental import pallas as pl
from jax.experimental.pallas import tpu as pltpu
from jax.experimental.pallas import tpu_sc as plsc

_DIAG_NO_GATHER = False
_DIAG_NO_SCATTER = False
_DIAG_TRIVIAL_COMPUTE = False

N_NODES = 100000
N_EDGES = 6400000
INV_TEMP = 10.0
NEG_INF = 1000000.0
STEPS = 10

NUM_TILES = 16
NODES_PAD = 102400
NODES_PER_TILE = NODES_PAD // NUM_TILES
EDGES_PER_TILE = N_EDGES // NUM_TILES
CHUNK = 16000
N_CHUNKS = EDGES_PER_TILE // CHUNK
U_INIT = 32.0

_LN2_HI = 0.693359375
_LN2_LO = -2.12194440e-4


def _log16(x):
    x = jnp.maximum(x, 1e-37)
    xi = lax.bitcast_convert_type(x, jnp.int32)
    e = ((xi >> 23) - 127).astype(jnp.float32)
    m = lax.bitcast_convert_type((xi & 0x007FFFFF) | 0x3F800000, jnp.float32)
    big = m > 1.41421356
    m = jnp.where(big, m * 0.5, m)
    e = jnp.where(big, e + 1.0, e)
    t = m - 1.0
    z = t * t
    p = jnp.full((16,), 7.0376836292e-2, jnp.float32)
    p = p * t + (-1.1514610310e-1)
    p = p * t + 1.1676998740e-1
    p = p * t + (-1.2420140846e-1)
    p = p * t + 1.4249322787e-1
    p = p * t + (-1.6668057665e-1)
    p = p * t + 2.0000714765e-1
    p = p * t + (-2.4999993993e-1)
    p = p * t + 3.3333331174e-1
    y = t * z * p - 0.5 * z
    r = e * _LN2_LO + y
    r = r + t
    return r + e * _LN2_HI


def _mesh():
    return plsc.VectorSubcoreMesh(
        core_axis_name="c", subcore_axis_name="s", num_cores=1,
        num_subcores=NUM_TILES)


@functools.partial(
    pl.kernel,
    out_type=jax.ShapeDtypeStruct((N_EDGES,), jnp.float32),
    mesh=_mesh(),
    scratch_types=[
        pltpu.VMEM_SHARED((NODES_PAD,), jnp.float32),
        pltpu.VMEM_SHARED((NODES_PAD,), jnp.float32),
        pltpu.VMEM_SHARED((NODES_PAD,), jnp.float32),
        pltpu.VMEM_SHARED((NODES_PAD,), jnp.float32),
        pltpu.VMEM((CHUNK,), jnp.float32),
        pltpu.VMEM((CHUNK,), jnp.int32),
        pltpu.VMEM((CHUNK,), jnp.int32),
        pltpu.VMEM((CHUNK,), jnp.float32),
        pltpu.VMEM((CHUNK,), jnp.float32),
        pltpu.VMEM((NODES_PER_TILE,), jnp.float32),
        pltpu.VMEM((NODES_PER_TILE,), jnp.float32),
        pltpu.VMEM((NODES_PER_TILE,), jnp.float32),
        pltpu.SemaphoreType.DMA,
    ],
)
def _sinkhorn(y_hbm, f_hbm, t_hbm, out_hbm,
              u, v, acc, acc2, y_b, f_b, t_b, uf_b, vt_b, a_b, a2_b,
              n_b, sem):
    wid = lax.axis_index("s")
    nsl = pl.ds(wid * NODES_PER_TILE, NODES_PER_TILE)

    @pl.loop(0, NODES_PER_TILE // 16)
    def _(i):
        sl = pl.ds(i * 16, 16)
        a_b[sl] = jnp.zeros((16,), jnp.float32)
        n_b[sl] = jnp.full((16,), U_INIT, jnp.float32)

    pltpu.sync_copy(n_b, u.at[nsl])
    pltpu.sync_copy(a_b, v.at[nsl])
    pltpu.sync_copy(a_b, acc.at[nsl])
    pltpu.sync_copy(a_b, acc2.at[nsl])
    plsc.subcore_barrier()

    def edge_pass(scatter: bool, is_row=None):
        @pl.loop(0, N_CHUNKS)
        def _(ci):
            base = wid * EDGES_PER_TILE + ci * CHUNK
            esl = pl.ds(base, CHUNK)
            pltpu.sync_copy(y_hbm.at[esl], y_b)
            pltpu.sync_copy(f_hbm.at[esl], f_b)
            pltpu.sync_copy(t_hbm.at[esl], t_b)
            if not _DIAG_NO_GATHER:
                pltpu.async_copy(u.at[f_b], uf_b, sem).wait()
                pltpu.async_copy(v.at[t_b], vt_b, sem).wait()

            @pl.loop(0, CHUNK // 16, unroll=4)
            def _(i):
                sl = pl.ds(i * 16, 16)
                fv = f_b[sl]
                tv = t_b[sl]
                zv = jnp.where(fv == tv, -NEG_INF, y_b[sl] * INV_TEMP)
                zv = zv - uf_b[sl] - vt_b[sl]
                y_b[sl] = jnp.exp(zv) if scatter else zv

            if scatter:
                odd = (wid & 1) == 1

                @pl.when(is_row & jnp.logical_not(odd))
                def _():
                    pltpu.sync_copy(y_b, acc.at[f_b], add=True)

                @pl.when(is_row & odd)
                def _():
                    pltpu.sync_copy(y_b, acc2.at[f_b], add=True)

                @pl.when(jnp.logical_not(is_row) & jnp.logical_not(odd))
                def _():
                    pltpu.sync_copy(y_b, acc.at[t_b], add=True)

                @pl.when(jnp.logical_not(is_row) & odd)
                def _():
                    pltpu.sync_copy(y_b, acc2.at[t_b], add=True)
            else:
                pltpu.sync_copy(y_b, out_hbm.at[esl])

    @pl.loop(0, 2 * STEPS)
    def _(step):
        is_row = (step & 1) == 0
        edge_pass(scatter=True, is_row=is_row)
        plsc.subcore_barrier()

        pltpu.sync_copy(acc.at[nsl], a_b)
        pltpu.sync_copy(acc2.at[nsl], a2_b)

        @pl.when(is_row)
        def _():
            pltpu.sync_copy(u.at[nsl], n_b)

        @pl.when(jnp.logical_not(is_row))
        def _():
            pltpu.sync_copy(v.at[nsl], n_b)

        @pl.loop(0, NODES_PER_TILE // 16)
        def _(i):
            sl = pl.ds(i * 16, 16)
            n_b[sl] = n_b[sl] + _log16(a_b[sl] + a2_b[sl])
            a_b[sl] = jnp.zeros((16,), jnp.float32)

        @pl.when(is_row)
        def _():
            pltpu.sync_copy(n_b, u.at[nsl])

        @pl.when(jnp.logical_not(is_row))
        def _():
            pltpu.sync_copy(n_b, v.at[nsl])

        pltpu.sync_copy(a_b, acc.at[nsl])
        pltpu.sync_copy(a_b, acc2.at[nsl])
        plsc.subcore_barrier()

    edge_pass(scatter=False)


def kernel(y, edge_index):
    return _sinkhorn(y, edge_index[0], edge_index[1])


# speedup vs baseline: 1.8555x; 1.8555x over previous
import functools

import jax
import jax.numpy as jnp
from jax import lax
from jax.experimental import pallas as pl
from jax.experimental.pallas import tpu as pltpu
from jax.experimental.pallas import tpu_sc as plsc

N_NODES = 100000
N_EDGES = 6400000
INV_TEMP = 10.0
NEG_INF = 1000000.0
STEPS = 10

NUM_CORES = 2
NUM_TILES = 16
NUM_WORKERS = NUM_CORES * NUM_TILES
NODES_PAD = 102400
NODES_PER_TILE = NODES_PAD // NUM_TILES
EDGES_PER_WORKER = N_EDGES // NUM_WORKERS
CHUNK = 10000
N_CHUNKS = EDGES_PER_WORKER // CHUNK
U_INIT = 32.0

_LN2_HI = 0.693359375
_LN2_LO = -2.12194440e-4

_F32 = jnp.float32


def _log16(x):
    x = jnp.maximum(x, 1e-37)
    xi = lax.bitcast_convert_type(x, jnp.int32)
    e = ((xi >> 23) - 127).astype(_F32)
    m = lax.bitcast_convert_type((xi & 0x007FFFFF) | 0x3F800000, _F32)
    big = m > 1.41421356
    m = jnp.where(big, m * 0.5, m)
    e = jnp.where(big, e + 1.0, e)
    t = m - 1.0
    z = t * t
    p = jnp.full((16,), 7.0376836292e-2, _F32)
    p = p * t + (-1.1514610310e-1)
    p = p * t + 1.1676998740e-1
    p = p * t + (-1.2420140846e-1)
    p = p * t + 1.4249322787e-1
    p = p * t + (-1.6668057665e-1)
    p = p * t + 2.0000714765e-1
    p = p * t + (-2.4999993993e-1)
    p = p * t + 3.3333331174e-1
    y = t * z * p - 0.5 * z
    r = e * _LN2_LO + y
    r = r + t
    return r + e * _LN2_HI


def _mesh():
    return plsc.VectorSubcoreMesh(
        core_axis_name="c", subcore_axis_name="s", num_cores=NUM_CORES,
        num_subcores=NUM_TILES)


def _scratch():
    return [
        pltpu.VMEM_SHARED((NODES_PAD,), _F32),
        pltpu.VMEM_SHARED((NODES_PAD,), _F32),
        pltpu.VMEM_SHARED((NODES_PAD,), _F32),
        pltpu.VMEM((CHUNK,), _F32),
        pltpu.VMEM((CHUNK,), jnp.int32),
        pltpu.VMEM((CHUNK,), jnp.int32),
        pltpu.VMEM((CHUNK,), _F32),
        pltpu.VMEM((CHUNK,), _F32),
        pltpu.VMEM((NODES_PER_TILE,), _F32),
        pltpu.VMEM((NODES_PER_TILE,), _F32),
        pltpu.VMEM((NODES_PER_TILE,), _F32),
        pltpu.SemaphoreType.DMA,
    ]


def _make_step(pend_is_u: bool, scatter_from: bool, first: bool,
               final: bool):
    if final:
        out_type = jax.ShapeDtypeStruct((N_EDGES,), _F32)
    else:
        out_type = (jax.ShapeDtypeStruct((NODES_PAD,), _F32),
                    jax.ShapeDtypeStruct((NODES_PAD,), _F32),
                    jax.ShapeDtypeStruct((NODES_PAD,), _F32))

    def body(y_hbm, f_hbm, t_hbm, *rest):
        if first:
            outs = rest[:3]
            scratch = rest[3:]
        elif final:
            pend_hbm, ready_hbm, aA_hbm, aB_hbm = rest[:4]
            outs = rest[4:5]
            scratch = rest[5:]
        else:
            pend_hbm, ready_hbm, aA_hbm, aB_hbm = rest[:4]
            outs = rest[4:7]
            scratch = rest[7:]
        (usp, vsp, acc, y_b, f_b, t_b, uf_b, vt_b, a_b, b_b, n_b,
         sem) = scratch
        cid = lax.axis_index("c")
        sid = lax.axis_index("s")
        wid = cid * NUM_TILES + sid
        nsl = pl.ds(sid * NODES_PER_TILE, NODES_PER_TILE)

        if not first:
            pltpu.sync_copy(pend_hbm.at[nsl], n_b)
            pltpu.sync_copy(aA_hbm.at[nsl], a_b)
            pltpu.sync_copy(aB_hbm.at[nsl], b_b)

        @pl.loop(0, NODES_PER_TILE // 16)
        def _(i):
            sl = pl.ds(i * 16, 16)
            if first:
                n_b[sl] = jnp.full((16,), U_INIT, _F32)
            else:
                n_b[sl] = n_b[sl] + _log16(a_b[sl] + b_b[sl])
            a_b[sl] = jnp.zeros((16,), _F32)

        pend_sp = usp if pend_is_u else vsp
        ready_sp = vsp if pend_is_u else usp
        pltpu.sync_copy(n_b, pend_sp.at[nsl])
        pltpu.sync_copy(a_b, acc.at[nsl])
        if first:
            pltpu.sync_copy(a_b, ready_sp.at[nsl])
        else:
            pltpu.sync_copy(ready_hbm.at[nsl], ready_sp.at[nsl])

            @pl.when(cid == 0)
            def _():
                pltpu.sync_copy(n_b, outs[0].at[nsl])

        plsc.subcore_barrier()

        @pl.loop(0, N_CHUNKS)
        def _(ci):
            esl = pl.ds(wid * EDGES_PER_WORKER + ci * CHUNK, CHUNK)
            pltpu.sync_copy(y_hbm.at[esl], y_b)
            pltpu.sync_copy(f_hbm.at[esl], f_b)
            pltpu.sync_copy(t_hbm.at[esl], t_b)
            pltpu.async_copy(usp.at[f_b], uf_b, sem).wait()
            pltpu.async_copy(vsp.at[t_b], vt_b, sem).wait()

            @pl.loop(0, CHUNK // 16, unroll=4)
            def _(i):
                sl = pl.ds(i * 16, 16)
                fv = f_b[sl]
                tv = t_b[sl]
                zv = jnp.where(fv == tv, -NEG_INF, y_b[sl] * INV_TEMP)
                zv = zv - uf_b[sl] - vt_b[sl]
                y_b[sl] = zv if final else jnp.exp(zv)

            if final:
                pltpu.sync_copy(y_b, outs[0].at[esl])
            elif scatter_from:
                pltpu.sync_copy(y_b, acc.at[f_b], add=True)
            else:
                pltpu.sync_copy(y_b, acc.at[t_b], add=True)

        if not final:
            plsc.subcore_barrier()

            @pl.when(cid == 0)
            def _():
                pltpu.sync_copy(acc.at[nsl], outs[1].at[nsl])

            @pl.when(cid == 1)
            def _():
                pltpu.sync_copy(acc.at[nsl], outs[2].at[nsl])

    return pl.kernel(body, out_type=out_type, mesh=_mesh(),
                     scratch_types=_scratch())


_step_first = _make_step(pend_is_u=True, scatter_from=True, first=True,
                         final=False)
_step_ut = _make_step(pend_is_u=True, scatter_from=False, first=False,
                      final=False)
_step_vf = _make_step(pend_is_u=False, scatter_from=True, first=False,
                      final=False)
_step_out = _make_step(pend_is_u=False, scatter_from=True, first=False,
                       final=True)


def kernel(y, edge_index):
    f = edge_index[0]
    t = edge_index[1]
    ub = jnp.full((NODES_PAD,), U_INIT, _F32)
    vb = jnp.zeros((NODES_PAD,), _F32)
    _, aA, aB = _step_first(y, f, t)
    for k in range(1, 2 * STEPS):
        if k % 2 == 1:
            ub, aA, aB = _step_ut(y, f, t, ub, vb, aA, aB)
        else:
            vb, aA, aB = _step_vf(y, f, t, vb, ub, aA, aB)
    return _step_out(y, f, t, vb, ub, aA, aB)


# speedup vs baseline: 2.0594x; 1.1099x over previous
import functools

import jax
import jax.numpy as jnp
from jax import lax
from jax.experimental import pallas as pl
from jax.experimental.pallas import tpu as pltpu
from jax.experimental.pallas import tpu_sc as plsc

N_NODES = 100000
N_EDGES = 6400000
INV_TEMP = 10.0
NEG_INF = 1000000.0
STEPS = 10

NUM_CORES = 2
NUM_TILES = 16
NUM_WORKERS = NUM_CORES * NUM_TILES
NODES_PAD = 102400
NODES_PER_TILE = NODES_PAD // NUM_TILES
EDGES_PER_WORKER = N_EDGES // NUM_WORKERS
CHUNK = 10000
N_CHUNKS = EDGES_PER_WORKER // CHUNK
U_INIT = 32.0

_LN2_HI = 0.693359375
_LN2_LO = -2.12194440e-4

_F32 = jnp.float32


def _log16(x):
    x = jnp.maximum(x, 1e-37)
    xi = lax.bitcast_convert_type(x, jnp.int32)
    e = ((xi >> 23) - 127).astype(_F32)
    m = lax.bitcast_convert_type((xi & 0x007FFFFF) | 0x3F800000, _F32)
    big = m > 1.41421356
    m = jnp.where(big, m * 0.5, m)
    e = jnp.where(big, e + 1.0, e)
    t = m - 1.0
    z = t * t
    p = jnp.full((16,), 7.0376836292e-2, _F32)
    p = p * t + (-1.1514610310e-1)
    p = p * t + 1.1676998740e-1
    p = p * t + (-1.2420140846e-1)
    p = p * t + 1.4249322787e-1
    p = p * t + (-1.6668057665e-1)
    p = p * t + 2.0000714765e-1
    p = p * t + (-2.4999993993e-1)
    p = p * t + 3.3333331174e-1
    y = t * z * p - 0.5 * z
    r = e * _LN2_LO + y
    r = r + t
    return r + e * _LN2_HI


def _mesh():
    return plsc.VectorSubcoreMesh(
        core_axis_name="c", subcore_axis_name="s", num_cores=NUM_CORES,
        num_subcores=NUM_TILES)


def _scratch():
    return [
        pltpu.VMEM_SHARED((NODES_PAD,), _F32),
        pltpu.VMEM_SHARED((NODES_PAD,), _F32),
        pltpu.VMEM_SHARED((NODES_PAD,), _F32),
        pltpu.VMEM((CHUNK,), _F32),
        pltpu.VMEM((CHUNK,), jnp.int32),
        pltpu.VMEM((CHUNK,), jnp.int32),
        pltpu.VMEM((CHUNK,), _F32),
        pltpu.VMEM((CHUNK,), _F32),
        pltpu.VMEM((CHUNK,), _F32),
        pltpu.VMEM((CHUNK,), _F32),
        pltpu.VMEM((CHUNK,), jnp.int32),
        pltpu.VMEM((CHUNK,), jnp.int32),
        pltpu.VMEM((NODES_PER_TILE,), _F32),
        pltpu.VMEM((NODES_PER_TILE,), _F32),
        pltpu.VMEM((NODES_PER_TILE,), _F32),
        pltpu.SemaphoreType.DMA,
        pltpu.SemaphoreType.DMA,
    ]


def _make_step(pend_is_u: bool, scatter_from: bool, first: bool,
               final: bool):
    if final:
        out_type = jax.ShapeDtypeStruct((N_EDGES,), _F32)
    else:
        out_type = (jax.ShapeDtypeStruct((NODES_PAD,), _F32),
                    jax.ShapeDtypeStruct((NODES_PAD,), _F32),
                    jax.ShapeDtypeStruct((NODES_PAD,), _F32))

    def body(y_hbm, f_hbm, t_hbm, *rest):
        if first:
            outs = rest[:3]
            scratch = rest[3:]
        elif final:
            pend_hbm, ready_hbm, aA_hbm, aB_hbm = rest[:4]
            outs = rest[4:5]
            scratch = rest[5:]
        else:
            pend_hbm, ready_hbm, aA_hbm, aB_hbm = rest[:4]
            outs = rest[4:7]
            scratch = rest[7:]
        (usp, vsp, acc, y_b, f_b, t_b, uf_b, vt_b, p0, p1, k0, k1,
         a_b, b_b, n_b, sem, ss) = scratch
        pbufs, kbufs = (p0, p1), (k0, k1)
        cid = lax.axis_index("c")
        sid = lax.axis_index("s")
        wid = cid * NUM_TILES + sid
        nsl = pl.ds(sid * NODES_PER_TILE, NODES_PER_TILE)

        if not first:
            pltpu.sync_copy(pend_hbm.at[nsl], n_b)
            pltpu.sync_copy(aA_hbm.at[nsl], a_b)
            pltpu.sync_copy(aB_hbm.at[nsl], b_b)

        @pl.loop(0, NODES_PER_TILE // 16)
        def _(i):
            sl = pl.ds(i * 16, 16)
            if first:
                n_b[sl] = jnp.full((16,), U_INIT, _F32)
            else:
                n_b[sl] = n_b[sl] + _log16(a_b[sl] + b_b[sl])
            a_b[sl] = jnp.zeros((16,), _F32)

        pend_sp = usp if pend_is_u else vsp
        ready_sp = vsp if pend_is_u else usp
        pltpu.sync_copy(n_b, pend_sp.at[nsl])
        pltpu.sync_copy(a_b, acc.at[nsl])
        if first:
            pltpu.sync_copy(a_b, ready_sp.at[nsl])
        else:
            pltpu.sync_copy(ready_hbm.at[nsl], ready_sp.at[nsl])

            @pl.when(cid == 0)
            def _():
                pltpu.sync_copy(n_b, outs[0].at[nsl])

        plsc.subcore_barrier()

        def drain():
            pltpu.make_async_copy(y_hbm.at[pl.ds(0, CHUNK)], p0, ss).wait()

        def chunk_body(ci, b):
            pb, kb = pbufs[b], kbufs[b]
            esl = pl.ds(wid * EDGES_PER_WORKER + ci * CHUNK, CHUNK)
            pltpu.sync_copy(y_hbm.at[esl], y_b)
            pltpu.sync_copy(f_hbm.at[esl], f_b)
            pltpu.sync_copy(t_hbm.at[esl], t_b)
            pltpu.async_copy(usp.at[f_b], uf_b, sem).wait()
            pltpu.async_copy(vsp.at[t_b], vt_b, sem).wait()

            @pl.loop(0, CHUNK // 16, unroll=4)
            def _(i):
                sl = pl.ds(i * 16, 16)
                fv = f_b[sl]
                tv = t_b[sl]
                zv = jnp.where(fv == tv, -NEG_INF, y_b[sl] * INV_TEMP)
                zv = zv - uf_b[sl] - vt_b[sl]
                pb[sl] = zv if final else jnp.exp(zv)
                if not final:
                    kb[sl] = fv if scatter_from else tv

            @pl.when(ci > 0)
            def _():
                drain()

            if final:
                pltpu.async_copy(pb, outs[0].at[esl], ss)
            else:
                pltpu.async_copy(pb, acc.at[kb], ss, add=True)

        @pl.loop(0, N_CHUNKS // 2)
        def _(g):
            chunk_body(g * 2, 0)
            chunk_body(g * 2 + 1, 1)

        drain()

        if not final:
            plsc.subcore_barrier()

            @pl.when(cid == 0)
            def _():
                pltpu.sync_copy(acc.at[nsl], outs[1].at[nsl])

            @pl.when(cid == 1)
            def _():
                pltpu.sync_copy(acc.at[nsl], outs[2].at[nsl])

    return pl.kernel(body, out_type=out_type, mesh=_mesh(),
                     scratch_types=_scratch())


_step_first = _make_step(pend_is_u=True, scatter_from=True, first=True,
                         final=False)
_step_ut = _make_step(pend_is_u=True, scatter_from=False, first=False,
                      final=False)
_step_vf = _make_step(pend_is_u=False, scatter_from=True, first=False,
                      final=False)
_step_out = _make_step(pend_is_u=False, scatter_from=True, first=False,
                       final=True)


def kernel(y, edge_index):
    f = edge_index[0]
    t = edge_index[1]
    ub = jnp.full((NODES_PAD,), U_INIT, _F32)
    vb = jnp.zeros((NODES_PAD,), _F32)
    _, aA, aB = _step_first(y, f, t)
    for k in range(1, 2 * STEPS):
        if k % 2 == 1:
            ub, aA, aB = _step_ut(y, f, t, ub, vb, aA, aB)
        else:
            vb, aA, aB = _step_vf(y, f, t, vb, ub, aA, aB)
    return _step_out(y, f, t, vb, ub, aA, aB)


# speedup vs baseline: 2.0803x; 1.0101x over previous
import functools

import jax
import jax.numpy as jnp
from jax import lax
from jax.experimental import pallas as pl
from jax.experimental.pallas import tpu as pltpu
from jax.experimental.pallas import tpu_sc as plsc

N_NODES = 100000
N_EDGES = 6400000
INV_TEMP = 10.0
NEG_INF = 1000000.0
STEPS = 10

NUM_CORES = 2
NUM_TILES = 16
NUM_WORKERS = NUM_CORES * NUM_TILES
NODES_PAD = 102400
NODES_PER_TILE = NODES_PAD // NUM_TILES
EDGES_PER_WORKER = N_EDGES // NUM_WORKERS
CHUNK = 10000
N_CHUNKS = EDGES_PER_WORKER // CHUNK
U_INIT = 32.0

_LN2_HI = 0.693359375
_LN2_LO = -2.12194440e-4

_F32 = jnp.float32


def _log16(x):
    x = jnp.maximum(x, 1e-37)
    xi = lax.bitcast_convert_type(x, jnp.int32)
    e = ((xi >> 23) - 127).astype(_F32)
    m = lax.bitcast_convert_type((xi & 0x007FFFFF) | 0x3F800000, _F32)
    big = m > 1.41421356
    m = jnp.where(big, m * 0.5, m)
    e = jnp.where(big, e + 1.0, e)
    t = m - 1.0
    z = t * t
    p = jnp.full((16,), 7.0376836292e-2, _F32)
    p = p * t + (-1.1514610310e-1)
    p = p * t + 1.1676998740e-1
    p = p * t + (-1.2420140846e-1)
    p = p * t + 1.4249322787e-1
    p = p * t + (-1.6668057665e-1)
    p = p * t + 2.0000714765e-1
    p = p * t + (-2.4999993993e-1)
    p = p * t + 3.3333331174e-1
    y = t * z * p - 0.5 * z
    r = e * _LN2_LO + y
    r = r + t
    return r + e * _LN2_HI


def _mesh():
    return plsc.VectorSubcoreMesh(
        core_axis_name="c", subcore_axis_name="s", num_cores=NUM_CORES,
        num_subcores=NUM_TILES)


def _scratch():
    return [
        pltpu.VMEM_SHARED((NODES_PAD,), _F32),
        pltpu.VMEM_SHARED((NODES_PAD,), _F32),
        pltpu.VMEM_SHARED((NODES_PAD,), _F32),
        pltpu.VMEM((CHUNK,), _F32),
        pltpu.VMEM((CHUNK,), jnp.int32),
        pltpu.VMEM((CHUNK,), jnp.int32),
        pltpu.VMEM((CHUNK,), _F32),
        pltpu.VMEM((CHUNK,), _F32),
        pltpu.VMEM((CHUNK,), _F32),
        pltpu.VMEM((CHUNK,), _F32),
        pltpu.VMEM((CHUNK,), jnp.int32),
        pltpu.VMEM((CHUNK,), jnp.int32),
        pltpu.VMEM((NODES_PER_TILE,), _F32),
        pltpu.VMEM((NODES_PER_TILE,), _F32),
        pltpu.VMEM((NODES_PER_TILE,), _F32),
        pltpu.SemaphoreType.DMA,
        pltpu.SemaphoreType.DMA,
        pltpu.SemaphoreType.DMA,
        pltpu.SemaphoreType.DMA,
        pltpu.SemaphoreType.DMA,
    ]


def _make_step(pend_is_u: bool, scatter_from: bool, first: bool,
               final: bool):
    if final:
        out_type = jax.ShapeDtypeStruct((N_EDGES,), _F32)
    else:
        out_type = (jax.ShapeDtypeStruct((NODES_PAD,), _F32),
                    jax.ShapeDtypeStruct((NODES_PAD,), _F32),
                    jax.ShapeDtypeStruct((NODES_PAD,), _F32))

    def body(y_hbm, f_hbm, t_hbm, *rest):
        if first:
            outs = rest[:3]
            scratch = rest[3:]
        elif final:
            pend_hbm, ready_hbm, aA_hbm, aB_hbm = rest[:4]
            outs = rest[4:5]
            scratch = rest[5:]
        else:
            pend_hbm, ready_hbm, aA_hbm, aB_hbm = rest[:4]
            outs = rest[4:7]
            scratch = rest[7:]
        (usp, vsp, acc, y_b, f_b, t_b, uf_b, vt_b, p0, p1, k0, k1,
         a_b, b_b, n_b, sem, ss, ly, lf, lt) = scratch
        pbufs, kbufs = (p0, p1), (k0, k1)
        cid = lax.axis_index("c")
        sid = lax.axis_index("s")
        wid = cid * NUM_TILES + sid
        nsl = pl.ds(sid * NODES_PER_TILE, NODES_PER_TILE)

        if not first:
            pltpu.sync_copy(pend_hbm.at[nsl], n_b)
            pltpu.sync_copy(aA_hbm.at[nsl], a_b)
            pltpu.sync_copy(aB_hbm.at[nsl], b_b)

        @pl.loop(0, NODES_PER_TILE // 16)
        def _(i):
            sl = pl.ds(i * 16, 16)
            if first:
                n_b[sl] = jnp.full((16,), U_INIT, _F32)
            else:
                n_b[sl] = n_b[sl] + _log16(a_b[sl] + b_b[sl])
            a_b[sl] = jnp.zeros((16,), _F32)

        pend_sp = usp if pend_is_u else vsp
        ready_sp = vsp if pend_is_u else usp
        pltpu.sync_copy(n_b, pend_sp.at[nsl])
        pltpu.sync_copy(a_b, acc.at[nsl])
        if first:
            pltpu.sync_copy(a_b, ready_sp.at[nsl])
        else:
            pltpu.sync_copy(ready_hbm.at[nsl], ready_sp.at[nsl])

            @pl.when(cid == 0)
            def _():
                pltpu.sync_copy(n_b, outs[0].at[nsl])

        plsc.subcore_barrier()

        def drain():
            pltpu.make_async_copy(y_hbm.at[pl.ds(0, CHUNK)], p0, ss).wait()

        def chunk_body(ci, b):
            pb, kb = pbufs[b], kbufs[b]
            esl = pl.ds(wid * EDGES_PER_WORKER + ci * CHUNK, CHUNK)
            pltpu.async_copy(f_hbm.at[esl], f_b, lf)
            pltpu.async_copy(t_hbm.at[esl], t_b, lt)
            pltpu.async_copy(y_hbm.at[esl], y_b, ly)
            pltpu.make_async_copy(f_hbm.at[esl], f_b, lf).wait()
            pltpu.make_async_copy(t_hbm.at[esl], t_b, lt).wait()
            g1 = pltpu.async_copy(usp.at[f_b], uf_b, sem)
            g2 = pltpu.async_copy(vsp.at[t_b], vt_b, sem)
            pltpu.make_async_copy(y_hbm.at[esl], y_b, ly).wait()
            g1.wait()
            g2.wait()

            @pl.loop(0, CHUNK // 16, unroll=4)
            def _(i):
                sl = pl.ds(i * 16, 16)
                fv = f_b[sl]
                tv = t_b[sl]
                zv = jnp.where(fv == tv, -NEG_INF, y_b[sl] * INV_TEMP)
                zv = zv - uf_b[sl] - vt_b[sl]
                pb[sl] = zv if final else jnp.exp(zv)
                if not final:
                    kb[sl] = fv if scatter_from else tv

            @pl.when(ci > 0)
            def _():
                drain()

            if final:
                pltpu.async_copy(pb, outs[0].at[esl], ss)
            else:
                pltpu.async_copy(pb, acc.at[kb], ss, add=True)

        @pl.loop(0, N_CHUNKS // 2)
        def _(g):
            chunk_body(g * 2, 0)
            chunk_body(g * 2 + 1, 1)

        drain()

        if not final:
            plsc.subcore_barrier()

            @pl.when(cid == 0)
            def _():
                pltpu.sync_copy(acc.at[nsl], outs[1].at[nsl])

            @pl.when(cid == 1)
            def _():
                pltpu.sync_copy(acc.at[nsl], outs[2].at[nsl])

    return pl.kernel(body, out_type=out_type, mesh=_mesh(),
                     scratch_types=_scratch())


_step_first = _make_step(pend_is_u=True, scatter_from=True, first=True,
                         final=False)
_step_ut = _make_step(pend_is_u=True, scatter_from=False, first=False,
                      final=False)
_step_vf = _make_step(pend_is_u=False, scatter_from=True, first=False,
                      final=False)
_step_out = _make_step(pend_is_u=False, scatter_from=True, first=False,
                       final=True)


def kernel(y, edge_index):
    f = edge_index[0]
    t = edge_index[1]
    ub = jnp.full((NODES_PAD,), U_INIT, _F32)
    vb = jnp.zeros((NODES_PAD,), _F32)
    _, aA, aB = _step_first(y, f, t)
    for k in range(1, 2 * STEPS):
        if k % 2 == 1:
            ub, aA, aB = _step_ut(y, f, t, ub, vb, aA, aB)
        else:
            vb, aA, aB = _step_vf(y, f, t, vb, ub, aA, aB)
    return _step_out(y, f, t, vb, ub, aA, aB)
